# baseline (speedup 1.0000x reference)
import jax
import jax.numpy as jnp
from jax import lax
from jax.experimental import pallas as pl
from jax.experimental.pallas import tpu as pltpu
from jax.experimental.pallas import tpu_sc as plsc

N = 10000
K = 16
F = 128
A = 128
DW = 16
NK = N * K
NCALLS = 6
NKALL = NCALLS * NK
NWORK = 32
CHUNK = 128
NBUF = 3

_HI = jax.lax.Precision.HIGHEST


def _make_sc_gather(nrows, width, untiled, pack8=False, zero_mask=False,
                    dep=False):
    rpw = nrows // NWORK
    nch = rpw // CHUNK
    tail = rpw - nch * CHUNK
    groups = nch // NBUF
    assert rpw * NWORK == nrows and groups * NBUF == nch and tail % 8 == 0
    pk = 128 // width if pack8 else 1
    out_shape = (nrows // pk, width * pk)

    def body(*refs):
        refs = list(refs)
        tab_hbm = refs.pop(0)
        idx_hbm = refs.pop(0)
        idxz_hbm = refs.pop(0) if zero_mask else None
        if dep:
            refs.pop(0)
        out_hbm = refs.pop(0)
        idx_v = refs.pop(0)
        idxz_v = refs.pop(0) if zero_mask else None
        r0, r1, r2 = refs[:3]
        refs = refs[3:]
        c = lax.axis_index("c")
        s = lax.axis_index("s")
        wid = s * 2 + c
        rbase = wid * rpw
        rb = [r0, r1, r2]
        wbf = refs[:3] if pk > 1 else rb
        sm = refs[3:] if pk > 1 else refs
        w0 = wbf[0]

        pltpu.sync_copy(idx_hbm.at[pl.ds(rbase, rpw)], idx_v)
        if zero_mask:
            pltpu.sync_copy(idxz_hbm.at[pl.ds(rbase, rpw)], idxz_v)

        def fire_gather(b, ch):
            pltpu.async_copy(
                tab_hbm.at[idx_v.at[pl.ds(ch * CHUNK, CHUNK)]], rb[b], sm[b])

        def wait_gather(b):
            pltpu.make_async_copy(
                tab_hbm.at[idx_v.at[pl.ds(0, CHUNK)]], rb[b], sm[b]).wait()

        def repack(b, ch, n=CHUNK):
            off = ch * CHUNK
            for i16 in range(n // 16):
                if zero_mask:
                    zv = idxz_v[pl.ds(off + i16 * 16, 16)]
                    mv = jnp.where(zv == 0, jnp.float32(0), jnp.float32(1))
                for j in range(16):
                    i = i16 * 16 + j
                    row = rb[b][i, :]
                    if zero_mask:
                        row = row * mv[j]
                    wbf[b][i // pk, pl.ds((i % pk) * width, width)] = row

        def fire_wb(b, ch):
            if pk > 1:
                repack(b, ch)
            pltpu.async_copy(
                wbf[b] if pk > 1 else rb[b],
                out_hbm.at[pl.ds((rbase + ch * CHUNK) // pk, CHUNK // pk)],
                sm[b])

        def wait_wb(b):
            src = wbf[b] if pk > 1 else rb[b]
            pltpu.make_async_copy(
                src, out_hbm.at[pl.ds(0, CHUNK // pk)], sm[b]).wait()

        for b in range(NBUF):
            fire_gather(b, b)

        def group(g, carry):
            ch0 = g * NBUF
            for b in range(NBUF):
                wait_gather(b)
                fire_wb(b, ch0 + b)
            for b in range(NBUF):
                @pl.when(g < groups - 1)
                def _(b=b):
                    wait_wb(b)
                    fire_gather(b, ch0 + NBUF + b)
            return carry

        lax.fori_loop(0, groups, group, 0)
        for b in range(NBUF):
            wait_wb(b)

        if tail:
            toff = nch * CHUNK
            t0 = r0.at[pl.ds(0, tail)]
            src = tab_hbm.at[idx_v.at[pl.ds(toff, tail)]]
            pltpu.async_copy(src, t0, sm[0])
            pltpu.make_async_copy(src, t0, sm[0]).wait()
            tsrc = r0 if pk == 1 else w0
            if pk > 1:
                for i16 in range(tail // 16):
                    if zero_mask:
                        zv = idxz_v[pl.ds(toff + i16 * 16, 16)]
                        mv = jnp.where(zv == 0, jnp.float32(0), jnp.float32(1))
                    for j in range(16):
                        i = i16 * 16 + j
                        row = r0[i, :]
                        if zero_mask:
                            row = row * mv[j]
                        w0[i // pk, pl.ds((i % pk) * width, width)] = row
            pltpu.sync_copy(
                tsrc.at[pl.ds(0, tail // pk)],
                out_hbm.at[pl.ds((rbase + toff) // pk, tail // pk)])

    params = pltpu.CompilerParams(use_tc_tiling_on_sc=False) if untiled else None

    def run(*args):
        return pl.kernel(
            body,
            mesh=plsc.VectorSubcoreMesh(core_axis_name="c", subcore_axis_name="s"),
            compiler_params=params,
            out_type=jax.ShapeDtypeStruct(out_shape, jnp.float32),
            scratch_types=(
                [pltpu.VMEM((rpw,), jnp.int32)] * (2 if zero_mask else 1)
                + [pltpu.VMEM((CHUNK, width), jnp.float32)] * 3
                + ([pltpu.VMEM((CHUNK // pk, width * pk), jnp.float32)] * 3
                   if pk > 1 else [])
                + [pltpu.SemaphoreType.DMA] * 3
            ),
        )(*args)

    return run


_gather_ej_call = _make_sc_gather(NK, F, untiled=False, dep=True)
_gather_ew_all = _make_sc_gather(NKALL, DW, untiled=True, pack8=True,
                                 zero_mask=True, dep=True)
_PK = 128 // DW


_B1 = 400


def _atten1_tc_body(ev_ref, ejn_ref, ewn_ref, w1e_ref, w1w_ref, w2_ref,
                    b_ref, v_ref, out_ref):
    bf = jnp.bfloat16
    f32 = jnp.float32
    ev = ev_ref[...]
    ejn = ejn_ref[...]
    ewp = ewn_ref[...]
    h = (jnp.dot(ev.astype(bf), w1e_ref[...].astype(bf),
                 preferred_element_type=f32) + b_ref[...])
    hj = jnp.dot(ejn.astype(bf), w2_ref[...].astype(bf),
                 preferred_element_type=f32)
    w1w = w1w_ref[...].astype(bf)
    ewpb = ewp.astype(bf)
    hps = [jnp.dot(ewpb[:, p * DW:(p + 1) * DW], w1w,
                   preferred_element_type=f32) for p in range(_PK)]
    hw = jnp.concatenate(hps, axis=0)
    av = (hj + hw).reshape(_B1, K, A) + h[:, None, :]
    x = jnp.sum(jnp.maximum(av, 0.0) * v_ref[...].reshape(1, 1, A), axis=-1)
    m = jnp.max(x, axis=1, keepdims=True)
    e = jnp.exp(x - m)
    a = e / jnp.sum(e, axis=1, keepdims=True)
    out_ref[...] = jnp.sum(a[:, :, None] * ejn.reshape(_B1, K, F), axis=1)


def _atten1_tc(call_idx, ev, ejn_c, ewn_all, w1e, w1w, w2, b, v):
    grid = (N // _B1,)
    nblk = NK // (_B1 * K)
    woff = call_idx * nblk
    return pl.pallas_call(
        _atten1_tc_body,
        grid=grid,
        in_specs=[
            pl.BlockSpec((_B1, F), lambda i: (i, 0)),
            pl.BlockSpec((_B1 * K, F), lambda i: (i, 0)),
            pl.BlockSpec((_B1 * K // _PK, DW * _PK), lambda i, o=woff: (o + i, 0)),
            pl.BlockSpec((F, A), lambda i: (0, 0)),
            pl.BlockSpec((DW, A), lambda i: (0, 0)),
            pl.BlockSpec((F, A), lambda i: (0, 0)),
            pl.BlockSpec((1, A), lambda i: (0, 0)),
            pl.BlockSpec((1, A), lambda i: (0, 0)),
        ],
        out_specs=pl.BlockSpec((_B1, F), lambda i: (i, 0)),
        out_shape=jax.ShapeDtypeStruct((N, F), jnp.float32),
    )(ev, ejn_c, ewn_all, w1e, w1w, w2, b, v)


def _atten2_tc_body(u_ref, i_ref, t_ref, U_ref, q_ref, p_ref, out_ref):
    u = u_ref[...]
    i = i_ref[...]
    t = t_ref[...]
    Um = U_ref[...]
    q = q_ref[...]
    p = p_ref[...]
    xu = jnp.dot(u, Um, precision=_HI) + q
    xi = jnp.dot(i, Um, precision=_HI) + q
    xt = jnp.dot(t, Um, precision=_HI) + q
    su = jnp.sum(jnp.maximum(xu, 0.0) * p, axis=-1, keepdims=True)
    si = jnp.sum(jnp.maximum(xi, 0.0) * p, axis=-1, keepdims=True)
    st = jnp.sum(jnp.maximum(xt, 0.0) * p, axis=-1, keepdims=True)
    x = jnp.concatenate([su, si, st], axis=1)
    m = jnp.max(x, axis=1, keepdims=True)
    e = jnp.exp(x - m)
    a = e / jnp.sum(e, axis=1, keepdims=True)
    out_ref[...] = (a[:, 0:1] * u + a[:, 1:2] * i + a[:, 2:3] * t)


def _atten2_tc(u, i, t, U, q, p):
    grid = (N // _B1,)
    blk = pl.BlockSpec((_B1, F), lambda g: (g, 0))
    return pl.pallas_call(
        _atten2_tc_body,
        grid=grid,
        in_specs=[blk, blk, blk,
                  pl.BlockSpec((F, A), lambda g: (0, 0)),
                  pl.BlockSpec((1, A), lambda g: (0, 0)),
                  pl.BlockSpec((1, A), lambda g: (0, 0))],
        out_specs=blk,
        out_shape=jax.ShapeDtypeStruct((N, F), jnp.float32),
    )(u, i, t, U, q, p)


def kernel(eu, ei, et, ew, u_iw_j, u_iw_w, u_tw_j, u_tw_w, i_uw_j, i_uw_w,
           i_tw_j, i_tw_w, t_uw_j, t_uw_w, t_iw_j, t_iw_w, W1_user, W2_user,
           b_user, v_user, W1_item, W2_item, b_item, v_item, W1_tag, W2_tag,
           b_tag, v_tag, U, q, p):
    zrow = jnp.zeros((1, F), jnp.float32)
    tabj = jnp.concatenate([zrow, eu, zrow, ei, zrow, et], axis=0)

    tmap = (1, 2, 0, 2, 0, 1)
    vjs = (u_iw_j, u_tw_j, i_uw_j, i_tw_j, t_uw_j, t_iw_j)
    vws = (u_iw_w, u_tw_w, i_uw_w, i_tw_w, t_uw_w, t_iw_w)
    offs = jnp.array([t * (N + 1) for t in tmap], jnp.int32)
    vj_all = (jnp.stack(vjs) + offs[:, None, None]).reshape(6, NK)
    nb = NKALL // (K * _B1)
    vw_perm = (jnp.stack(vws).reshape(nb, _PK, K * _B1 // _PK)
               .transpose(0, 2, 1).reshape(-1))
    vw_g = jnp.maximum(vw_perm - 1, 0)

    ejns = [_gather_ej_call(tabj, vj_all[0], offs)]
    ewn_all = _gather_ew_all(ew, vw_g, vw_perm, ejns[0])
    prev = ewn_all
    for c in range(1, 6):
        ejns.append(_gather_ej_call(tabj, vj_all[c], prev))
        prev = ejns[c]

    def atten1(c, ev, W1, W2, b, v):
        return _atten1_tc(c, ev, ejns[c], ewn_all,
                          W1[:F], W1[F:], W2, b, v)

    eu_iN = atten1(0, eu, W1_item, W2_item, b_item, v_item)
    eu_tN = atten1(1, eu, W1_tag, W2_tag, b_tag, v_tag)
    ei_uN = atten1(2, ei, W1_user, W2_user, b_user, v_user)
    ei_tN = atten1(3, ei, W1_tag, W2_tag, b_tag, v_tag)
    et_uN = atten1(4, et, W1_user, W2_user, b_user, v_user)
    et_iN = atten1(5, et, W1_item, W2_item, b_item, v_item)

    euN = _atten2_tc(eu, eu_iN, eu_tN, U, q, p)
    eiN = _atten2_tc(ei_uN, ei, ei_tN, U, q, p)
    etN = _atten2_tc(et_uN, et_iN, et, U, q, p)
    return (euN, eiN, etN)


# speedup vs baseline: 1.1053x; 1.1053x over previous
import jax
import jax.numpy as jnp
from jax import lax
from jax.experimental import pallas as pl
from jax.experimental.pallas import tpu as pltpu
from jax.experimental.pallas import tpu_sc as plsc

N = 10000
K = 16
F = 128
A = 128
DW = 16
NK = N * K
NCALLS = 6
NKALL = NCALLS * NK
NWORK = 32
CHUNK = 128
NBUF = 3

_HI = jax.lax.Precision.HIGHEST


def _make_sc_gather(nrows, width, untiled, pack8=False, zero_mask=False,
                    dep=False):
    rpw = nrows // NWORK
    nch = rpw // CHUNK
    tail = rpw - nch * CHUNK
    groups = nch // NBUF
    assert rpw * NWORK == nrows and groups * NBUF == nch and tail % 8 == 0
    pk = 128 // width if pack8 else 1
    out_shape = (nrows // pk, width * pk)

    def body(*refs):
        refs = list(refs)
        tab_hbm = refs.pop(0)
        idx_hbm = refs.pop(0)
        idxz_hbm = refs.pop(0) if zero_mask else None
        if dep:
            refs.pop(0)
        out_hbm = refs.pop(0)
        idx_v = refs.pop(0)
        idxz_v = refs.pop(0) if zero_mask else None
        r0, r1, r2 = refs[:3]
        refs = refs[3:]
        c = lax.axis_index("c")
        s = lax.axis_index("s")
        wid = s * 2 + c
        rbase = wid * rpw
        rb = [r0, r1, r2]
        wbf = refs[:3] if pk > 1 else rb
        sm = refs[3:] if pk > 1 else refs
        w0 = wbf[0]

        pltpu.sync_copy(idx_hbm.at[pl.ds(rbase, rpw)], idx_v)
        if zero_mask:
            pltpu.sync_copy(idxz_hbm.at[pl.ds(rbase, rpw)], idxz_v)

        def fire_gather(b, ch):
            pltpu.async_copy(
                tab_hbm.at[idx_v.at[pl.ds(ch * CHUNK, CHUNK)]], rb[b], sm[b])

        def wait_gather(b):
            pltpu.make_async_copy(
                tab_hbm.at[idx_v.at[pl.ds(0, CHUNK)]], rb[b], sm[b]).wait()

        def repack(b, ch, n=CHUNK):
            off = ch * CHUNK
            for i16 in range(n // 16):
                if zero_mask:
                    zv = idxz_v[pl.ds(off + i16 * 16, 16)]
                    mv = jnp.where(zv == 0, jnp.float32(0), jnp.float32(1))
                for j in range(16):
                    i = i16 * 16 + j
                    row = rb[b][i, :]
                    if zero_mask:
                        row = row * mv[j]
                    wbf[b][i // pk, pl.ds((i % pk) * width, width)] = row

        def fire_wb(b, ch):
            if pk > 1:
                repack(b, ch)
            pltpu.async_copy(
                wbf[b] if pk > 1 else rb[b],
                out_hbm.at[pl.ds((rbase + ch * CHUNK) // pk, CHUNK // pk)],
                sm[b])

        def wait_wb(b):
            src = wbf[b] if pk > 1 else rb[b]
            pltpu.make_async_copy(
                src, out_hbm.at[pl.ds(0, CHUNK // pk)], sm[b]).wait()

        for b in range(NBUF):
            fire_gather(b, b)

        def group(g, carry):
            ch0 = g * NBUF
            for b in range(NBUF):
                wait_gather(b)
                fire_wb(b, ch0 + b)
            for b in range(NBUF):
                @pl.when(g < groups - 1)
                def _(b=b):
                    wait_wb(b)
                    fire_gather(b, ch0 + NBUF + b)
            return carry

        lax.fori_loop(0, groups, group, 0)
        for b in range(NBUF):
            wait_wb(b)

        if tail:
            toff = nch * CHUNK
            t0 = r0.at[pl.ds(0, tail)]
            src = tab_hbm.at[idx_v.at[pl.ds(toff, tail)]]
            pltpu.async_copy(src, t0, sm[0])
            pltpu.make_async_copy(src, t0, sm[0]).wait()
            tsrc = r0 if pk == 1 else w0
            if pk > 1:
                for i16 in range(tail // 16):
                    if zero_mask:
                        zv = idxz_v[pl.ds(toff + i16 * 16, 16)]
                        mv = jnp.where(zv == 0, jnp.float32(0), jnp.float32(1))
                    for j in range(16):
                        i = i16 * 16 + j
                        row = r0[i, :]
                        if zero_mask:
                            row = row * mv[j]
                        w0[i // pk, pl.ds((i % pk) * width, width)] = row
            pltpu.sync_copy(
                tsrc.at[pl.ds(0, tail // pk)],
                out_hbm.at[pl.ds((rbase + toff) // pk, tail // pk)])

    params = pltpu.CompilerParams(use_tc_tiling_on_sc=False) if untiled else None

    def run(*args):
        return pl.kernel(
            body,
            mesh=plsc.VectorSubcoreMesh(core_axis_name="c", subcore_axis_name="s"),
            compiler_params=params,
            out_type=jax.ShapeDtypeStruct(out_shape, jnp.float32),
            scratch_types=(
                [pltpu.VMEM((rpw,), jnp.int32)] * (2 if zero_mask else 1)
                + [pltpu.VMEM((CHUNK, width), jnp.float32)] * 3
                + ([pltpu.VMEM((CHUNK // pk, width * pk), jnp.float32)] * 3
                   if pk > 1 else [])
                + [pltpu.SemaphoreType.DMA] * 3
            ),
        )(*args)

    return run


_gather_ej_half = _make_sc_gather(NKALL // 2, F, untiled=False, dep=True)
_gather_ew_all = _make_sc_gather(NKALL, DW, untiled=True, pack8=True,
                                 zero_mask=True, dep=True)
_PK = 128 // DW


_B1 = 400


def _atten1_tc_body(ev_ref, ejn_ref, ewn_ref, w1e_ref, w1w_ref, w2_ref,
                    b_ref, v_ref, out_ref):
    bf = jnp.bfloat16
    f32 = jnp.float32
    ev = ev_ref[...]
    ejn = ejn_ref[...]
    ewp = ewn_ref[...]
    h = (jnp.dot(ev.astype(bf), w1e_ref[...].astype(bf),
                 preferred_element_type=f32) + b_ref[...])
    hj = jnp.dot(ejn.astype(bf), w2_ref[...].astype(bf),
                 preferred_element_type=f32)
    w1w = w1w_ref[...].astype(bf)
    ewpb = ewp.astype(bf)
    hps = [jnp.dot(ewpb[:, p * DW:(p + 1) * DW], w1w,
                   preferred_element_type=f32) for p in range(_PK)]
    hw = jnp.concatenate(hps, axis=0)
    av = (hj + hw).reshape(_B1, K, A) + h[:, None, :]
    x = jnp.sum(jnp.maximum(av, 0.0) * v_ref[...].reshape(1, 1, A), axis=-1)
    m = jnp.max(x, axis=1, keepdims=True)
    e = jnp.exp(x - m)
    a = e / jnp.sum(e, axis=1, keepdims=True)
    out_ref[...] = jnp.sum(a[:, :, None] * ejn.reshape(_B1, K, F), axis=1)


def _atten1_tc(call_idx, ev, ejn_c, ewn_all, w1e, w1w, w2, b, v):
    grid = (N // _B1,)
    nblk = NK // (_B1 * K)
    joff = (call_idx % 3) * nblk
    woff = call_idx * nblk
    return pl.pallas_call(
        _atten1_tc_body,
        grid=grid,
        in_specs=[
            pl.BlockSpec((_B1, F), lambda i: (i, 0)),
            pl.BlockSpec((_B1 * K, F), lambda i, o=joff: (o + i, 0)),
            pl.BlockSpec((_B1 * K // _PK, DW * _PK), lambda i, o=woff: (o + i, 0)),
            pl.BlockSpec((F, A), lambda i: (0, 0)),
            pl.BlockSpec((DW, A), lambda i: (0, 0)),
            pl.BlockSpec((F, A), lambda i: (0, 0)),
            pl.BlockSpec((1, A), lambda i: (0, 0)),
            pl.BlockSpec((1, A), lambda i: (0, 0)),
        ],
        out_specs=pl.BlockSpec((_B1, F), lambda i: (i, 0)),
        out_shape=jax.ShapeDtypeStruct((N, F), jnp.float32),
    )(ev, ejn_c, ewn_all, w1e, w1w, w2, b, v)


def _atten2_tc_body(u_ref, i_ref, t_ref, U_ref, q_ref, p_ref, out_ref):
    u = u_ref[...]
    i = i_ref[...]
    t = t_ref[...]
    Um = U_ref[...]
    q = q_ref[...]
    p = p_ref[...]
    xu = jnp.dot(u, Um, precision=_HI) + q
    xi = jnp.dot(i, Um, precision=_HI) + q
    xt = jnp.dot(t, Um, precision=_HI) + q
    su = jnp.sum(jnp.maximum(xu, 0.0) * p, axis=-1, keepdims=True)
    si = jnp.sum(jnp.maximum(xi, 0.0) * p, axis=-1, keepdims=True)
    st = jnp.sum(jnp.maximum(xt, 0.0) * p, axis=-1, keepdims=True)
    x = jnp.concatenate([su, si, st], axis=1)
    m = jnp.max(x, axis=1, keepdims=True)
    e = jnp.exp(x - m)
    a = e / jnp.sum(e, axis=1, keepdims=True)
    out_ref[...] = (a[:, 0:1] * u + a[:, 1:2] * i + a[:, 2:3] * t)


def _atten2_tc(u, i, t, U, q, p):
    grid = (N // _B1,)
    blk = pl.BlockSpec((_B1, F), lambda g: (g, 0))
    return pl.pallas_call(
        _atten2_tc_body,
        grid=grid,
        in_specs=[blk, blk, blk,
                  pl.BlockSpec((F, A), lambda g: (0, 0)),
                  pl.BlockSpec((1, A), lambda g: (0, 0)),
                  pl.BlockSpec((1, A), lambda g: (0, 0))],
        out_specs=blk,
        out_shape=jax.ShapeDtypeStruct((N, F), jnp.float32),
    )(u, i, t, U, q, p)


def kernel(eu, ei, et, ew, u_iw_j, u_iw_w, u_tw_j, u_tw_w, i_uw_j, i_uw_w,
           i_tw_j, i_tw_w, t_uw_j, t_uw_w, t_iw_j, t_iw_w, W1_user, W2_user,
           b_user, v_user, W1_item, W2_item, b_item, v_item, W1_tag, W2_tag,
           b_tag, v_tag, U, q, p):
    zrow = jnp.zeros((1, F), jnp.float32)
    tabj = jnp.concatenate([zrow, eu, zrow, ei, zrow, et], axis=0)

    tmap = (1, 2, 0, 2, 0, 1)
    vjs = (u_iw_j, u_tw_j, i_uw_j, i_tw_j, t_uw_j, t_iw_j)
    vws = (u_iw_w, u_tw_w, i_uw_w, i_tw_w, t_uw_w, t_iw_w)
    offs = jnp.array([t * (N + 1) for t in tmap], jnp.int32)
    vj_all = (jnp.stack(vjs) + offs[:, None, None]).reshape(2, 3 * NK)
    nb = NKALL // (K * _B1)
    vw_perm = (jnp.stack(vws).reshape(nb, _PK, K * _B1 // _PK)
               .transpose(0, 2, 1).reshape(-1))
    vw_g = jnp.maximum(vw_perm - 1, 0)

    ejn_h0 = _gather_ej_half(tabj, vj_all[0], offs)
    ewn_all = _gather_ew_all(ew, vw_g, vw_perm, ejn_h0)
    ejn_h1 = _gather_ej_half(tabj, vj_all[1], ewn_all)
    ejn_halves = (ejn_h0, ejn_h1)

    def atten1(c, ev, W1, W2, b, v):
        return _atten1_tc(c, ev, ejn_halves[c // 3], ewn_all,
                          W1[:F], W1[F:], W2, b, v)

    eu_iN = atten1(0, eu, W1_item, W2_item, b_item, v_item)
    eu_tN = atten1(1, eu, W1_tag, W2_tag, b_tag, v_tag)
    ei_uN = atten1(2, ei, W1_user, W2_user, b_user, v_user)
    ei_tN = atten1(3, ei, W1_tag, W2_tag, b_tag, v_tag)
    et_uN = atten1(4, et, W1_user, W2_user, b_user, v_user)
    et_iN = atten1(5, et, W1_item, W2_item, b_item, v_item)

    euN = _atten2_tc(eu, eu_iN, eu_tN, U, q, p)
    eiN = _atten2_tc(ei_uN, ei, ei_tN, U, q, p)
    etN = _atten2_tc(et_uN, et_iN, et, U, q, p)
    return (euN, eiN, etN)
